# baseline (speedup 1.0000x reference)
import functools

import jax
import jax.numpy as jnp
from jax import lax
from jax.experimental import pallas as pl
from jax.experimental.pallas import tpu as pltpu
from jax.experimental.pallas import tpu_sc as plsc

_CLS = 100000
_B = 1024
_SMOOTH = 0.1
_CONF = 1.0 - _SMOOTH
_EPS = _SMOOTH / (_CLS - 2)
_BC = 2048
_NB = (_CLS + _BC - 1) // _BC

_NW = 32
_BPW = _B // _NW


def _sc_gather_body(pred_flat, tgt_hbm, out_hbm, t_v, idx_v, val_v, sem):
    wid = lax.axis_index("s") * 2 + lax.axis_index("c")
    base = wid * _BPW
    pltpu.sync_copy(tgt_hbm.at[pl.ds(base, _BPW)], t_v)
    for k in range(_BPW // 16):
        rows = (base + k * 16) + lax.iota(jnp.int32, 16)
        flat = rows * _CLS + t_v[pl.ds(k * 16, 16)]
        idx_v[pl.ds(k * 16, 16)] = flat
    pltpu.async_copy(pred_flat.at[idx_v], val_v, sem).wait()
    pltpu.sync_copy(val_v, out_hbm.at[pl.ds(base, _BPW)])


_sc_gather = functools.partial(
    pl.kernel,
    mesh=plsc.VectorSubcoreMesh(core_axis_name="c", subcore_axis_name="s"),
    out_type=jax.ShapeDtypeStruct((_B,), jnp.float32),
    scratch_types=[
        pltpu.VMEM((_BPW,), jnp.int32),
        pltpu.VMEM((_BPW,), jnp.int32),
        pltpu.VMEM((_BPW,), jnp.float32),
        pltpu.SemaphoreType.DMA,
    ],
)(_sc_gather_body)


def _stats_body(pred_ref, m_ref, s_ref, sp_ref, p0_ref, av_ref, ai_ref):
    j = pl.program_id(0)

    @pl.when(j == 0)
    def _init():
        neg = jnp.full((_B, 1), -jnp.inf, jnp.float32)
        m_ref[...] = neg
        av_ref[...] = neg
        s_ref[...] = jnp.zeros((_B, 1), jnp.float32)
        sp_ref[...] = jnp.zeros((_B, 1), jnp.float32)
        ai_ref[...] = jnp.zeros((_B, 1), jnp.int32)

    x = pred_ref[...]
    licol = jax.lax.broadcasted_iota(jnp.int32, (_B, _BC), 1)

    def _update(xm, xz):
        bmax = jnp.max(xm, axis=1, keepdims=True)
        m_old = m_ref[...]
        m_new = jnp.maximum(m_old, bmax)
        s_ref[...] = (s_ref[...] * jnp.exp(m_old - m_new)
                      + jnp.sum(jnp.exp(xm - m_new), axis=1, keepdims=True))
        m_ref[...] = m_new
        sp_ref[...] += jnp.sum(xz, axis=1, keepdims=True)
        bidx = jnp.min(jnp.where(xm == bmax, licol, jnp.int32(2**30)),
                       axis=1, keepdims=True)
        better = bmax > av_ref[...]
        av_ref[...] = jnp.where(better, bmax, av_ref[...])
        ai_ref[...] = jnp.where(better, bidx + j * _BC, ai_ref[...])

    @pl.when(j < _NB - 1)
    def _hot():
        _update(x, x)

    @pl.when(j == _NB - 1)
    def _tail():
        valid = licol < (_CLS - (_NB - 1) * _BC)
        _update(jnp.where(valid, x, -jnp.inf), jnp.where(valid, x, 0.0))

    @pl.when(j == 0)
    def _p0():
        p0_ref[...] = x[:, 0:1]


def _epi_body(m_ref, s_ref, sp_ref, p0_ref, ai_ref, tv_ref, t_ref, o_ref):
    lse = m_ref[...] + jnp.log(s_ref[...])
    tv = tv_ref[...]
    t = t_ref[...]
    ce_row = lse - _CONF * tv - _EPS * (sp_ref[...] - p0_ref[...] - tv)
    ce_row = jnp.where(t == 0, 0.0, ce_row)
    ce = jnp.sum(ce_row) * (1.0 / _B)
    d = 1.0 - jnp.exp(-ce)
    f_loss = d * d * ce

    ai = ai_ref[...]
    cp = jnp.where(ai < 5000, ai % 100, -1)
    ct = jnp.where(t < 5000, t % 100, -1)
    pen = jnp.where(ai == t, 0.0, jnp.where(cp == ct, 0.5, 1.0))
    cc = jnp.sum(pen) * (1.0 / _B)
    o_ref[...] = jnp.broadcast_to(f_loss + cc, (1, 1))


@jax.jit
def kernel(pred, target):
    t32 = target.astype(jnp.int32)
    t2 = t32.reshape(_B, 1)
    tv = _sc_gather(pred.reshape(_B * _CLS), t32)
    col = pl.BlockSpec((_B, 1), lambda j: (0, 0))
    stats = pl.pallas_call(
        _stats_body,
        grid=(_NB,),
        in_specs=[pl.BlockSpec((_B, _BC), lambda j: (0, j))],
        out_specs=[col] * 6,
        out_shape=[jax.ShapeDtypeStruct((_B, 1), jnp.float32)] * 5
        + [jax.ShapeDtypeStruct((_B, 1), jnp.int32)],
    )(pred)
    m, s, sp, p0, _av, ai = stats
    out = pl.pallas_call(
        _epi_body,
        out_shape=jax.ShapeDtypeStruct((1, 1), jnp.float32),
    )(m, s, sp, p0, ai, tv.reshape(_B, 1), t2)
    return out.reshape(())
